# Initial kernel scaffold; baseline (speedup 1.0000x reference)
import jax
import jax.numpy as jnp
from jax.experimental import pallas as pl


def kernel(x, edge_index, edge_attr, W, b):
    raise NotImplementedError("write your pallas kernel here")


# speedup vs baseline: 38.0360x; 38.0360x over previous
import functools

import jax
import jax.numpy as jnp
from jax import lax
from jax.experimental import pallas as pl
from jax.experimental.pallas import tpu as pltpu
from jax.experimental.pallas import tpu_sc as plsc

N_NODES = 10000
N_EDGES = 320000
D_FEAT = 128

NC, NS, L = 2, 16, 16
NW = NC * NS
E_PER_W = N_EDGES // NW
STEPS = E_PER_W // L


def _matvec_body(x_ref, w_ref, y_ref):
    y_ref[...] = jnp.sum(x_ref[...] * w_ref[...], axis=1, keepdims=True)


def _combine_body(p_ref, b_ref, o_ref):
    o_ref[...] = jnp.sum(p_ref[...]) * (1.0 / N_NODES) + b_ref[...]


@functools.cache
def _edge_reduce_kernel():
    mesh = plsc.VectorSubcoreMesh(core_axis_name="c", subcore_axis_name="s")

    @functools.partial(
        pl.kernel,
        mesh=mesh,
        compiler_params=pltpu.CompilerParams(needs_layout_passes=False),
        out_type=jax.ShapeDtypeStruct((NW, L), jnp.float32),
        scratch_types=[
            pltpu.VMEM((N_NODES,), jnp.float32),
            pltpu.VMEM((E_PER_W,), jnp.int32),
            pltpu.VMEM((E_PER_W,), jnp.float32),
            pltpu.VMEM((L,), jnp.float32),
        ],
    )
    def _edge_reduce(y_hbm, src_hbm, attr_hbm, out_hbm, y_v, src_v, attr_v, acc_v):
        wid = lax.axis_index("s") * NC + lax.axis_index("c")
        base = wid * E_PER_W
        pltpu.sync_copy(y_hbm, y_v)
        pltpu.sync_copy(src_hbm.at[pl.ds(base, E_PER_W)], src_v)
        pltpu.sync_copy(attr_hbm.at[pl.ds(base, E_PER_W)], attr_v)

        def body(i, acc):
            idx = src_v[pl.ds(i * L, L)]
            a = attr_v[pl.ds(i * L, L)]
            g = plsc.load_gather(y_v, [idx])
            return acc + a * g

        acc = lax.fori_loop(0, STEPS, body, jnp.zeros((L,), jnp.float32))
        acc_v[...] = acc
        pltpu.sync_copy(acc_v, out_hbm.at[wid])

    return _edge_reduce


def kernel(x, edge_index, edge_attr, W, b):
    src = edge_index[0].astype(jnp.int32)
    y2d = pl.pallas_call(
        _matvec_body,
        out_shape=jax.ShapeDtypeStruct((N_NODES, 1), jnp.float32),
    )(x, W)
    y = y2d.reshape(N_NODES)
    partials = _edge_reduce_kernel()(y, src, edge_attr)
    out = pl.pallas_call(
        _combine_body,
        out_shape=jax.ShapeDtypeStruct((1, 1), jnp.float32),
    )(partials, b.reshape(1, 1))
    return out


# speedup vs baseline: 43.7715x; 1.1508x over previous
import functools

import jax
import jax.numpy as jnp
from jax import lax
from jax.experimental import pallas as pl
from jax.experimental.pallas import tpu as pltpu
from jax.experimental.pallas import tpu_sc as plsc

N_NODES = 10000
N_EDGES = 320000
D_FEAT = 128

NC, NS, L = 2, 16, 16
NW = NC * NS
E_PER_W = N_EDGES // NW
STEPS = E_PER_W // L
N_VECS = N_NODES // L


@functools.cache
def _scatter_w_kernel():
    mesh = plsc.VectorSubcoreMesh(core_axis_name="c", subcore_axis_name="s")

    @functools.partial(
        pl.kernel,
        mesh=mesh,
        compiler_params=pltpu.CompilerParams(needs_layout_passes=False),
        out_type=jax.ShapeDtypeStruct((NW, N_NODES), jnp.float32),
        scratch_types=[
            pltpu.VMEM((N_NODES,), jnp.float32),
            pltpu.VMEM((E_PER_W,), jnp.int32),
            pltpu.VMEM((E_PER_W,), jnp.float32),
        ],
    )
    def _scatter_w(src_hbm, attr_hbm, out_hbm, acc_v, src_v, attr_v):
        wid = lax.axis_index("s") * NC + lax.axis_index("c")
        base = wid * E_PER_W
        pltpu.sync_copy(src_hbm.at[pl.ds(base, E_PER_W)], src_v)
        pltpu.sync_copy(attr_hbm.at[pl.ds(base, E_PER_W)], attr_v)

        zeros = jnp.zeros((L,), jnp.float32)

        def zero_body(i, _):
            acc_v[pl.ds(i * L, L)] = zeros
            return 0

        lax.fori_loop(0, N_VECS, zero_body, 0)

        def body(i, _):
            idx = src_v[pl.ds(i * L, L)]
            a = attr_v[pl.ds(i * L, L)]
            plsc.addupdate_scatter(acc_v, [idx], a)
            return 0

        lax.fori_loop(0, STEPS, body, 0)
        pltpu.sync_copy(acc_v, out_hbm.at[wid])

    return _scatter_w


def _dense_body(p_ref, x_ref, w_ref, b_ref, o_ref):
    w_nodes = jnp.sum(p_ref[...], axis=0, keepdims=True)
    v = jax.lax.dot_general(
        w_nodes, x_ref[...], (((1,), (0,)), ((), ())),
        precision=jax.lax.Precision.HIGHEST,
        preferred_element_type=jnp.float32)
    o_ref[...] = jnp.sum(v * w_ref[...]) * (1.0 / N_NODES) + b_ref[...]


def kernel(x, edge_index, edge_attr, W, b):
    src = edge_index[0].astype(jnp.int32)
    partials = _scatter_w_kernel()(src, edge_attr)
    out = pl.pallas_call(
        _dense_body,
        out_shape=jax.ShapeDtypeStruct((1, 1), jnp.float32),
    )(partials, x, W, b.reshape(1, 1))
    return out


# speedup vs baseline: 60.0316x; 1.3715x over previous
import functools

import jax
import jax.numpy as jnp
from jax import lax
from jax.experimental import pallas as pl
from jax.experimental.pallas import tpu as pltpu
from jax.experimental.pallas import tpu_sc as plsc

N_NODES = 10000
N_EDGES = 320000
D_FEAT = 128

NC, NS, L = 2, 16, 16
NW = NC * NS
E_PER_W = N_EDGES // NW
STEPS = E_PER_W // L
N_VECS = N_NODES // L
E_BUF = 10240


@functools.cache
def _scatter_w_kernel():
    mesh = plsc.VectorSubcoreMesh(core_axis_name="c", subcore_axis_name="s")

    @functools.partial(
        pl.kernel,
        mesh=mesh,
        compiler_params=pltpu.CompilerParams(needs_layout_passes=False),
        out_type=jax.ShapeDtypeStruct((NW, N_NODES), jnp.float32),
        scratch_types=[
            pltpu.VMEM((N_NODES,), jnp.float32),
            pltpu.VMEM((2, E_BUF), jnp.int32),
            pltpu.VMEM((E_PER_W,), jnp.float32),
        ],
    )
    def _scatter_w(edge_hbm, attr_hbm, out_hbm, acc_v, src_v, attr_v):
        wid = lax.axis_index("s") * NC + lax.axis_index("c")
        base = wid * E_PER_W
        base_al = pl.multiple_of(
            jnp.minimum((base // 128) * 128, N_EDGES - E_BUF), 128)
        off = base - base_al
        pltpu.sync_copy(edge_hbm.at[:, pl.ds(base_al, E_BUF)], src_v)
        pltpu.sync_copy(attr_hbm.at[pl.ds(base, E_PER_W)], attr_v)

        zeros = jnp.zeros((L,), jnp.float32)

        def zero_body(i, _):
            acc_v[pl.ds(i * L, L)] = zeros
            return 0

        lax.fori_loop(0, N_VECS, zero_body, 0)

        def body(i, _):
            idx = src_v[0, pl.ds(off + i * L, L)]
            a = attr_v[pl.ds(i * L, L)]
            plsc.addupdate_scatter(acc_v, [idx], a)
            return 0

        lax.fori_loop(0, STEPS, body, 0)
        pltpu.sync_copy(acc_v, out_hbm.at[wid])

    return _scatter_w


def _dense_body(p_ref, x_ref, w_ref, b_ref, o_ref):
    w_nodes = jnp.sum(p_ref[...], axis=0, keepdims=True)
    v = jax.lax.dot_general(
        w_nodes, x_ref[...], (((1,), (0,)), ((), ())),
        precision=jax.lax.Precision.HIGHEST,
        preferred_element_type=jnp.float32)
    o_ref[...] = jnp.sum(v * w_ref[...]) * (1.0 / N_NODES) + b_ref[...]


def kernel(x, edge_index, edge_attr, W, b):
    partials = _scatter_w_kernel()(edge_index.astype(jnp.int32), edge_attr)
    out = pl.pallas_call(
        _dense_body,
        out_shape=jax.ShapeDtypeStruct((1, 1), jnp.float32),
    )(partials, x, W, b.reshape(1, 1))
    return out
